# Initial kernel scaffold; baseline (speedup 1.0000x reference)
import jax
import jax.numpy as jnp
from jax.experimental import pallas as pl
from jax.experimental.pallas import tpu as pltpu

_INV_T = 100.0
_L, _P, _D = 16, 768, 256
_N = _L * _P
_BI = 128
_CORES = 2
_GI = _N // (_BI * _CORES)
_JC = 1024


def _matmul_body(lm_ref, pf_ref, out_ref):
    out_ref[...] = jax.lax.dot_general(
        lm_ref[...], pf_ref[...],
        (((1,), (1,)), ((), ())),
        preferred_element_type=jnp.float32,
    )


def _main_body(k_ref, s_col_ref, pos_col_ref, s_row_ref, w_ref, out_ref):
    c = pl.program_id(0)
    g = pl.program_id(1)

    @pl.when(g == 0)
    def _init():
        out_ref[...] = jnp.zeros_like(out_ref)

    blk = g * _CORES + c

    @pl.when(blk * _BI < k_ref[0])
    def _compute():
        s_col = s_col_ref[...]
        acc = jnp.zeros((_BI, 2), dtype=jnp.float32)
        for jc in range(_N // _JC):
            s_row = s_row_ref[:, jc * _JC:(jc + 1) * _JC]
            d = jax.nn.sigmoid((s_row - s_col) * _INV_T)
            w = w_ref[jc * _JC:(jc + 1) * _JC, :]
            acc = acc + jax.lax.dot_general(
                d, w, (((1,), (0,)), ((), ())),
                preferred_element_type=jnp.float32,
            )
        rpn = 1.0 + acc[:, 0:1]
        rp = 1.0 + acc[:, 1:2]
        pos_col = pos_col_ref[...]
        num = jnp.sum(pos_col * rp / rpn)
        npos = jnp.sum(pos_col)
        lane = jax.lax.broadcasted_iota(jnp.int32, (1, 128), 1)
        out_ref[...] += jnp.where(
            lane == 0, num, jnp.where(lane == 1, npos, 0.0))


def kernel(landmark_embeddings, patch_features, pos_patches, pos_neg_patches):
    sims = pl.pallas_call(
        _matmul_body,
        out_shape=jax.ShapeDtypeStruct((_L, _P), jnp.float32),
    )(landmark_embeddings, patch_features)

    s_flat = sims.reshape(-1)
    pos_f = pos_patches.reshape(-1).astype(jnp.float32)
    pn_f = pos_neg_patches.reshape(-1).astype(jnp.float32)

    order = jnp.argsort(jnp.where(pos_patches.reshape(-1), 0, 1))
    s_col = s_flat[order].reshape(_N, 1)
    pos_col = pos_f[order].reshape(_N, 1)
    k_count = jnp.sum(pos_patches).astype(jnp.int32).reshape(1)

    s_row = s_flat.reshape(1, _N)
    w = jnp.stack([pn_f, pn_f * pos_f], axis=1)

    out = pl.pallas_call(
        _main_body,
        grid=(_CORES, _GI),
        in_specs=[
            pl.BlockSpec(memory_space=pltpu.SMEM),
            pl.BlockSpec((_BI, 1), lambda c, g: (g * _CORES + c, 0)),
            pl.BlockSpec((_BI, 1), lambda c, g: (g * _CORES + c, 0)),
            pl.BlockSpec((1, _N), lambda c, g: (0, 0)),
            pl.BlockSpec((_N, 2), lambda c, g: (0, 0)),
        ],
        out_specs=pl.BlockSpec((1, 128), lambda c, g: (c, 0)),
        out_shape=jax.ShapeDtypeStruct((_CORES, 128), jnp.float32),
        compiler_params=pltpu.CompilerParams(
            dimension_semantics=("parallel", "arbitrary"),
        ),
    )(k_count, s_col, pos_col, s_row, w)

    num = out[0, 0] + out[1, 0]
    npos = out[0, 1] + out[1, 1]
    return -(num / npos)


# speedup vs baseline: 3.2038x; 3.2038x over previous
import jax
import jax.numpy as jnp
from jax.experimental import pallas as pl
from jax.experimental.pallas import tpu as pltpu

_INV_T = 100.0
_L, _P, _D = 16, 768, 256
_N = _L * _P
_BI = 128
_CORES = 2
_GI = _N // (_BI * _CORES)
_JC = 1024


def _matmul_body(lm_ref, pf_ref, out_ref):
    out_ref[...] = jax.lax.dot_general(
        lm_ref[...], pf_ref[...],
        (((1,), (1,)), ((), ())),
        preferred_element_type=jnp.float32,
    )


def _main_body(k_ref, s_col_ref, pos_col_ref, s_row_ref, w_ref, out_ref):
    c = pl.program_id(0)
    g = pl.program_id(1)

    @pl.when(g == 0)
    def _init():
        out_ref[...] = jnp.zeros_like(out_ref)

    blk = g * _CORES + c

    @pl.when(blk * _BI < k_ref[0])
    def _compute():
        s_col = s_col_ref[...]
        acc = jnp.zeros((_BI, 2), dtype=jnp.float32)
        for jc in range(_N // _JC):
            s_row = s_row_ref[:, jc * _JC:(jc + 1) * _JC]
            d = jax.nn.sigmoid((s_row - s_col) * _INV_T)
            w = w_ref[jc * _JC:(jc + 1) * _JC, :]
            acc = acc + jax.lax.dot_general(
                d, w, (((1,), (0,)), ((), ())),
                preferred_element_type=jnp.float32,
            )
        rpn = 1.0 + acc[:, 0:1]
        rp = 1.0 + acc[:, 1:2]
        pos_col = pos_col_ref[...]
        num = jnp.sum(pos_col * rp / rpn)
        npos = jnp.sum(pos_col)
        lane = jax.lax.broadcasted_iota(jnp.int32, (1, 1, 128), 2)
        out_ref[...] += jnp.where(
            lane == 0, num, jnp.where(lane == 1, npos, 0.0))


def kernel(landmark_embeddings, patch_features, pos_patches, pos_neg_patches):
    sims = pl.pallas_call(
        _matmul_body,
        out_shape=jax.ShapeDtypeStruct((_L, _P), jnp.float32),
    )(landmark_embeddings, patch_features)

    s_flat = sims.reshape(-1)
    pos_f = pos_patches.reshape(-1).astype(jnp.float32)
    pn_f = pos_neg_patches.reshape(-1).astype(jnp.float32)

    order = jnp.argsort(jnp.where(pos_patches.reshape(-1), 0, 1))
    s_col = s_flat[order].reshape(_N, 1)
    pos_col = pos_f[order].reshape(_N, 1)
    k_count = jnp.sum(pos_patches).astype(jnp.int32).reshape(1)

    s_row = s_flat.reshape(1, _N)
    w = jnp.stack([pn_f, pn_f * pos_f], axis=1)

    out = pl.pallas_call(
        _main_body,
        grid=(_CORES, _GI),
        in_specs=[
            pl.BlockSpec(memory_space=pltpu.SMEM),
            pl.BlockSpec((_BI, 1), lambda c, g: (g * _CORES + c, 0)),
            pl.BlockSpec((_BI, 1), lambda c, g: (g * _CORES + c, 0)),
            pl.BlockSpec((1, _N), lambda c, g: (0, 0)),
            pl.BlockSpec((_N, 2), lambda c, g: (0, 0)),
        ],
        out_specs=pl.BlockSpec((1, 1, 128), lambda c, g: (c, 0, 0)),
        out_shape=jax.ShapeDtypeStruct((_CORES, 1, 128), jnp.float32),
        compiler_params=pltpu.CompilerParams(
            dimension_semantics=("parallel", "arbitrary"),
        ),
    )(k_count, s_col, pos_col, s_row, w)

    num = out[0, 0, 0] + out[1, 0, 0]
    npos = out[0, 0, 1] + out[1, 0, 1]
    return -(num / npos)


# speedup vs baseline: 10.4304x; 3.2557x over previous
import jax
import jax.numpy as jnp
from jax.experimental import pallas as pl
from jax.experimental.pallas import tpu as pltpu

_INV_T = 100.0
_L, _P, _D = 16, 768, 256
_N = _L * _P
_BI = 128
_NB = _N // _BI
_CORES = 2
_JC = 1024


def _matmul_body(lm_ref, pf_ref, out_ref):
    out_ref[...] = jax.lax.dot_general(
        lm_ref[...], pf_ref[...],
        (((1,), (1,)), ((), ())),
        preferred_element_type=jnp.float32,
    )


def _main_body(k_ref, s_mat_ref, pos_mat_ref, s_row_ref, w_ref, out_ref):
    c = pl.program_id(0)
    nb = (k_ref[0] + (_BI - 1)) // _BI
    trips = (nb + 1 - c) // _CORES

    def body(b, carry):
        num_acc, npos_acc = carry
        blk = b * _CORES + c
        s_i_row = s_mat_ref[pl.ds(blk, 1), 0, :]
        pos_row = pos_mat_ref[pl.ds(blk, 1), 0, :]
        both = jnp.concatenate([s_i_row, pos_row], axis=0)
        both_t = jax.lax.transpose(both, (1, 0))
        s_col = both_t[:, 0:1]
        pos_col = both_t[:, 1:2]

        acc = jnp.zeros((_BI, 2), dtype=jnp.float32)
        for jc in range(_N // _JC):
            s_row = s_row_ref[:, jc * _JC:(jc + 1) * _JC]
            d = jax.nn.sigmoid((s_row - s_col) * _INV_T)
            w = w_ref[:, jc * _JC:(jc + 1) * _JC]
            acc = acc + jax.lax.dot_general(
                d, w, (((1,), (1,)), ((), ())),
                preferred_element_type=jnp.float32,
            )
        rpn = 1.0 + acc[:, 0:1]
        rp = 1.0 + acc[:, 1:2]
        num_acc = num_acc + jnp.sum(pos_col * rp / rpn)
        npos_acc = npos_acc + jnp.sum(pos_col)
        return num_acc, npos_acc

    num, npos = jax.lax.fori_loop(
        0, trips, body, (jnp.float32(0.0), jnp.float32(0.0)))
    lane = jax.lax.broadcasted_iota(jnp.int32, (1, 1, 128), 2)
    out_ref[...] = jnp.where(lane == 0, num, jnp.where(lane == 1, npos, 0.0))


def kernel(landmark_embeddings, patch_features, pos_patches, pos_neg_patches):
    sims = pl.pallas_call(
        _matmul_body,
        out_shape=jax.ShapeDtypeStruct((_L, _P), jnp.float32),
    )(landmark_embeddings, patch_features)

    s_flat = sims.reshape(-1)
    pos_b = pos_patches.reshape(-1)
    pos_f = pos_b.astype(jnp.float32)
    pn_f = pos_neg_patches.reshape(-1).astype(jnp.float32)

    key = jnp.where(pos_b, jnp.int32(0), jnp.int32(1))
    _, s_perm, pos_perm = jax.lax.sort(
        (key, s_flat, pos_f), dimension=0, num_keys=1)
    s_mat = s_perm.reshape(_NB, 1, _BI)
    pos_mat = pos_perm.reshape(_NB, 1, _BI)
    k_count = jnp.sum(pos_b).astype(jnp.int32).reshape(1)

    s_row = s_flat.reshape(1, _N)
    w = jnp.stack([pn_f, pn_f * pos_f], axis=0)

    out = pl.pallas_call(
        _main_body,
        grid=(_CORES,),
        in_specs=[
            pl.BlockSpec(memory_space=pltpu.SMEM),
            pl.BlockSpec((_NB, 1, _BI), lambda c: (0, 0, 0)),
            pl.BlockSpec((_NB, 1, _BI), lambda c: (0, 0, 0)),
            pl.BlockSpec((1, _N), lambda c: (0, 0)),
            pl.BlockSpec((2, _N), lambda c: (0, 0)),
        ],
        out_specs=pl.BlockSpec((1, 1, 128), lambda c: (c, 0, 0)),
        out_shape=jax.ShapeDtypeStruct((_CORES, 1, 128), jnp.float32),
        compiler_params=pltpu.CompilerParams(
            dimension_semantics=("parallel",),
        ),
    )(k_count, s_mat, pos_mat, s_row, w)

    num = out[0, 0, 0] + out[1, 0, 0]
    npos = out[0, 0, 1] + out[1, 0, 1]
    return -(num / npos)
